# baseline (speedup 1.0000x reference)
import functools

import jax
import jax.numpy as jnp
from jax import lax
from jax.experimental import pallas as pl
from jax.experimental.pallas import tpu as pltpu
from jax.experimental.pallas import tpu_sc as plsc

N_ATOMS = 1048576
N_EMB = 128

NC = 2
NS = 16
NW = NC * NS

G = N_ATOMS // 128
GPW = G // NW
OUTER = GPW // 8
CHUNK = 16
NCHUNK = GPW // CHUNK
NBUF_D = 4
NBUF_V = 2
P_TOT = OUTER * 4


def _sc_gather(table, x2d):
    mesh = plsc.VectorSubcoreMesh(core_axis_name="c", subcore_axis_name="s")

    @functools.partial(
        pl.kernel,
        mesh=mesh,
        out_type=jax.ShapeDtypeStruct((G, 128, N_EMB), jnp.float32),
        scratch_types=[
            pltpu.VMEM((2, CHUNK, 128), jnp.int32),
            pltpu.VMEM((NBUF_D, 128, N_EMB), jnp.float32),
            pltpu.VMEM((NBUF_V, 128, N_EMB), jnp.float32),
            pltpu.VMEM((100, N_EMB), jnp.float32),
            pltpu.VMEM_SHARED((100, N_EMB), jnp.float32),
            pltpu.SemaphoreType.DMA,
            pltpu.SemaphoreType.DMA,
            pltpu.SemaphoreType.DMA,
            pltpu.SemaphoreType.DMA,
        ],
    )
    def body(table_hbm, idx_hbm, out_hbm, idx_c, rows_d, rows_v2, table_loc,
             table_sh, sem_gd, sem_sd, sem_sv, sem_i):
        wid = lax.axis_index("s") * NC + lax.axis_index("c")
        base = wid * GPW

        @pl.when(lax.axis_index("s") == 0)
        def _():
            pltpu.sync_copy(table_hbm, table_sh)

        pltpu.sync_copy(table_hbm, table_loc)
        pltpu.sync_copy(idx_hbm.at[pl.ds(base, CHUNK)], idx_c.at[0])
        pltpu.async_copy(idx_hbm.at[pl.ds(base + CHUNK, CHUNK)],
                         idx_c.at[1], sem_i)
        plsc.subcore_barrier()

        def idx_ref(l):
            return idx_c.at[(l // CHUNK) % 2, l % CHUNK]

        def idx_fetch(ck):
            pltpu.async_copy(idx_hbm.at[pl.ds(base + ck * CHUNK, CHUNK)],
                             idx_c.at[ck % 2], sem_i)

        def idx_wait(ck):
            pltpu.make_async_copy(
                idx_hbm.at[pl.ds(base + ck * CHUNK, CHUNK)],
                idx_c.at[ck % 2], sem_i).wait()

        def gd(l, b):
            pltpu.async_copy(table_sh.at[idx_ref(l)], rows_d.at[b], sem_gd)

        def gd_wait(l, b):
            pltpu.make_async_copy(
                table_sh.at[idx_ref(l)], rows_d.at[b], sem_gd).wait()

        def sd(l, b):
            pltpu.async_copy(rows_d.at[b], out_hbm.at[base + l], sem_sd)

        def sd_wait(l, b):
            pltpu.make_async_copy(
                rows_d.at[b], out_hbm.at[base + l], sem_sd).wait()

        def sv(l, b):
            pltpu.async_copy(rows_v2.at[b], out_hbm.at[base + l], sem_sv)

        def sv_wait(l, b):
            pltpu.make_async_copy(
                rows_v2.at[b], out_hbm.at[base + l], sem_sv).wait()

        def fill(l, b):
            lrow = idx_ref(l)

            def rbody(rb, carry):
                iv = lrow[pl.ds(rb * 16, 16)]
                ro = rb * 16
                for k in range(16):
                    r = iv[k]
                    for c in range(8):
                        rows_v2[b, ro + k, pl.ds(c * 16, 16)] = (
                            table_loc[r, pl.ds(c * 16, 16)])
                return carry

            lax.fori_loop(0, 8, rbody, 0)

        gd(0, 0)
        gd(1, 1)

        def step(i, carry):
            @pl.when((i % 2 == 1) & (i <= 2 * NCHUNK - 3))
            def _():
                idx_wait((i + 1) // 2)

            for u in range(4):
                p = i * 4 + u
                l = i * 8 + u
                gd_wait(l, u)
                sd(l, u)

                @pl.when(p >= 2)
                def _():
                    sd_wait(l - 2 if u >= 2 else l - 6, (u + 2) % NBUF_D)

                @pl.when(p + 2 < P_TOT)
                def _():
                    gd(l + 2 if u < 2 else l + 6, (u + 2) % NBUF_D)

                v = i * 4 + u
                lv = i * 8 + 4 + u

                @pl.when(v >= 2)
                def _():
                    sv_wait(lv - 2 if u >= 2 else lv - 6, u % NBUF_V)

                fill(lv, u % NBUF_V)
                sv(lv, u % NBUF_V)

            @pl.when((i % 2 == 1) & (i <= 2 * NCHUNK - 5))
            def _():
                idx_fetch((i + 3) // 2)

            return carry

        lax.fori_loop(0, OUTER, step, 0)

        last = (OUTER - 1) * 8
        sd_wait(last + 2, 2)
        sd_wait(last + 3, 3)
        sv_wait(last + 6, 0)
        sv_wait(last + 7, 1)

    return body(table, x2d)


def kernel(x, embedding_list):
    out = _sc_gather(embedding_list, x.reshape(G, 128))
    return out.reshape(N_ATOMS, N_EMB)


# speedup vs baseline: 2.1459x; 2.1459x over previous
import functools

import jax
import jax.numpy as jnp
from jax import lax
from jax.experimental import pallas as pl
from jax.experimental.pallas import tpu as pltpu
from jax.experimental.pallas import tpu_sc as plsc

N_ATOMS = 1048576
N_EMB = 128

NC = 2
NS = 16
NW = NC * NS

G = N_ATOMS // 128
GPW = G // NW
NBUF = 4
UNROLL = NBUF


def _sc_gather(table, x2d):
    mesh = plsc.VectorSubcoreMesh(core_axis_name="c", subcore_axis_name="s")

    @functools.partial(
        pl.kernel,
        mesh=mesh,
        out_type=jax.ShapeDtypeStruct((G, 128, N_EMB), jnp.float32),
        scratch_types=[
            pltpu.VMEM((GPW, 128), jnp.int32),
            pltpu.VMEM((NBUF, 128, N_EMB), jnp.float32),
            pltpu.VMEM_SHARED((NS * 100, N_EMB), jnp.float32),
            pltpu.SemaphoreType.DMA,
            pltpu.SemaphoreType.DMA,
        ],
    )
    def body(table_hbm, idx_hbm, out_hbm, idx_v, rows_v, table_sh, sem_g, sem_s):
        wid = lax.axis_index("s") * NC + lax.axis_index("c")
        base = wid * GPW

        sid = lax.axis_index("s")

        @pl.when(sid == 0)
        def _():
            for r in range(NS):
                pltpu.sync_copy(table_hbm, table_sh.at[pl.ds(r * 100, 100)])

        plsc.subcore_barrier()

        def gather(t, b):
            pltpu.async_copy(table_sh.at[idx_v.at[t]], rows_v.at[b], sem_g)

        def gather_wait(t, b):
            pltpu.make_async_copy(table_sh.at[idx_v.at[t]], rows_v.at[b], sem_g).wait()

        def scatter(t, b):
            pltpu.async_copy(rows_v.at[b], out_hbm.at[base + t], sem_s)

        def scatter_wait(t, b):
            pltpu.make_async_copy(rows_v.at[b], out_hbm.at[base + t], sem_s).wait()

        pltpu.sync_copy(idx_hbm.at[pl.ds(base, GPW)], idx_v)
        bias = jnp.full((16,), 0, jnp.int32) + sid * 100

        def bias_row(rr, carry):
            for cc in range(8):
                idx_v[rr, pl.ds(cc * 16, 16)] = (
                    idx_v[rr, pl.ds(cc * 16, 16)] + bias)
            return carry

        lax.fori_loop(0, GPW, bias_row, 0)

        gather(0, 0)
        gather(1, 1)

        def step(i, carry):
            for u in range(UNROLL):
                t = i * UNROLL + u
                gather_wait(t, u)
                scatter(t, u)

                @pl.when(t >= 2)
                def _():
                    scatter_wait(t - 2, (u + 2) % NBUF)

                @pl.when(t + 2 < GPW)
                def _():
                    gather(t + 2, (u + 2) % NBUF)
            return carry

        lax.fori_loop(0, GPW // UNROLL, step, 0)

        scatter_wait(GPW - 2, (GPW - 2) % NBUF)
        scatter_wait(GPW - 1, (GPW - 1) % NBUF)

    return body(table, x2d)


def kernel(x, embedding_list):
    out = _sc_gather(embedding_list, x.reshape(G, 128))
    return out.reshape(N_ATOMS, N_EMB)
